# baseline (speedup 1.0000x reference)
import functools

import jax
import jax.numpy as jnp
from jax import lax
from jax.experimental import pallas as pl
from jax.experimental.pallas import tpu as pltpu
from jax.experimental.pallas import tpu_sc as plsc

_B = 16384
_C = 1000
_SC_SAMPLES = 4096
_B1 = _B - _SC_SAMPLES
_BL = 2048
_LPW = _SC_SAMPLES // 32


def _tc_body(xt_ref, y_ref, loss_ref, *, n_classes, bl):
    ch = 8
    y = y_ref[...]
    sub = jax.lax.broadcasted_iota(jnp.int32, (ch, bl), 0)
    d = y[None, :] - sub

    acc_y = jnp.zeros((ch, bl), jnp.float32)
    for k in range(n_classes // ch):
        xk = xt_ref[k * ch:(k + 1) * ch, :]
        acc_y = acc_y + jnp.where(d == ch * k, xk, 0.0)
    t = jnp.sum(acc_y, axis=0) - 1.0

    acc_s = jnp.zeros((ch, bl), jnp.float32)
    for k in range(n_classes // ch):
        xk = xt_ref[k * ch:(k + 1) * ch, :]
        acc_s = acc_s + jnp.maximum(xk - t[None, :], 0.0)
    s = jnp.sum(acc_s, axis=0)

    loss_ref[...] = (s - 1.0) * (1.0 / n_classes)


def _sc_body(xt_hbm, y_hbm, out_hbm, xbuf, ybuf, obuf, sem, ysem, osem):
    wid = lax.axis_index("s") * 2 + lax.axis_index("c")
    l0 = _B1 + wid * _LPW
    pltpu.async_copy(y_hbm.at[pl.ds(l0, _LPW)], ybuf, ysem).wait()
    pltpu.async_copy(xt_hbm.at[:, pl.ds(l0, _LPW)], xbuf, sem).wait()

    lanes = lax.iota(jnp.int32, 16)
    for g in range(_LPW // 16):
        yg = ybuf[pl.ds(g * 16, 16)]
        oy = plsc.load_gather(xbuf, [yg, g * 16 + lanes])
        t = oy - 1.0

        def rbody(r, acc, g=g, t=t):
            for s in range(8):
                v = xbuf[r * 8 + s, pl.ds(g * 16, 16)]
                acc = acc + jnp.maximum(v, t)
            return acc

        s_sum = lax.fori_loop(0, _C // 8, rbody, jnp.zeros(16, jnp.float32))
        obuf[pl.ds(g * 16, 16)] = (s_sum - _C * t - 1.0) * (1.0 / _C)
    pltpu.async_copy(obuf, out_hbm.at[pl.ds(wid * _LPW, _LPW)], osem).wait()


def _sc_call(xt, y):
    mesh = plsc.VectorSubcoreMesh(
        core_axis_name="c", subcore_axis_name="s", num_cores=2, num_subcores=16
    )
    f = pl.kernel(
        _sc_body,
        out_type=jax.ShapeDtypeStruct((_SC_SAMPLES,), jnp.float32),
        mesh=mesh,
        scratch_types=[
            pltpu.VMEM((_C, _LPW), jnp.float32),
            pltpu.VMEM((_LPW,), jnp.int32),
            pltpu.VMEM((_LPW,), jnp.float32),
            pltpu.SemaphoreType.DMA,
            pltpu.SemaphoreType.DMA,
            pltpu.SemaphoreType.DMA,
        ],
        compiler_params=pltpu.CompilerParams(needs_layout_passes=False),
    )
    return f(xt, y)


def kernel(output, y):
    b, c = output.shape
    y = y.astype(jnp.int32)
    xt = output.T
    tc_body = functools.partial(_tc_body, n_classes=c, bl=_BL)
    tc_loss = pl.pallas_call(
        tc_body,
        grid=(_B1 // _BL,),
        in_specs=[
            pl.BlockSpec((c, _BL), lambda i: (0, i)),
            pl.BlockSpec((_BL,), lambda i: (i,)),
        ],
        out_specs=pl.BlockSpec((_BL,), lambda i: (i,)),
        out_shape=jax.ShapeDtypeStruct((_B1,), jnp.float32),
    )(xt, y)
    sc_loss = _sc_call(xt, y)
    return jnp.concatenate([tc_loss, sc_loss])


# speedup vs baseline: 1.8825x; 1.8825x over previous
import functools

import jax
import jax.numpy as jnp
from jax.experimental import pallas as pl


def _hinge_body(xt_ref, y_ref, loss_ref, *, n_classes, bl):
    ch = 8
    y = y_ref[...]
    sub = jax.lax.broadcasted_iota(jnp.int32, (ch, bl), 0)
    d = y[None, :] - sub

    acc_y = jnp.zeros((ch, bl), jnp.float32)
    for k in range(n_classes // ch):
        xk = xt_ref[k * ch:(k + 1) * ch, :]
        acc_y = acc_y + jnp.where(d == ch * k, xk, 0.0)
    t = jnp.sum(acc_y, axis=0) - 1.0

    acc_s = jnp.zeros((ch, bl), jnp.float32)
    for k in range(n_classes // ch):
        xk = xt_ref[k * ch:(k + 1) * ch, :]
        acc_s = acc_s + jnp.maximum(xk - t[None, :], 0.0)
    s = jnp.sum(acc_s, axis=0)

    loss_ref[...] = (s - 1.0) * (1.0 / n_classes)


def kernel(output, y):
    b, c = output.shape
    y = y.astype(jnp.int32)
    xt = output.T
    bl = 2048
    grid = (b // bl,)
    body = functools.partial(_hinge_body, n_classes=c, bl=bl)
    return pl.pallas_call(
        body,
        grid=grid,
        in_specs=[
            pl.BlockSpec((c, bl), lambda i: (0, i)),
            pl.BlockSpec((bl,), lambda i: (i,)),
        ],
        out_specs=pl.BlockSpec((bl,), lambda i: (i,)),
        out_shape=jax.ShapeDtypeStruct((b,), jnp.float32),
    )(xt, y)
